# baseline (speedup 1.0000x reference)
import math

import jax
import jax.numpy as jnp
import numpy as np
from jax import lax
from jax.experimental import pallas as pl
from jax.experimental.pallas import tpu as pltpu

_NUM_BINS = 8
_NCH = 16
_GL = _NCH * _NUM_BINS
_LEFT = -1.0
_SPAN = 2.0
_MINW = 1e-4
_MIND = 1e-4
_BLIM = _LEFT + 1e-3
_ULIM = -_LEFT - 1e-3
_DCONST = math.log(math.exp(1.0 - _MIND) - 1.0)
_PD = lax.Precision.DEFAULT


def _softplus(z):
    return jnp.maximum(z, 0.0) + jnp.log(1.0 + jnp.exp(-jnp.abs(z)))


def _dot01(a, m):
    hi = a.astype(jnp.bfloat16)
    lo = (a - hi.astype(jnp.float32)).astype(jnp.bfloat16)
    return (jnp.dot(hi, m, preferred_element_type=jnp.float32, precision=_PD)
            + jnp.dot(lo, m, preferred_element_type=jnp.float32, precision=_PD))


def _grp_cumsum(x, kf):
    s = x + jnp.where(kf >= 1.0, pltpu.roll(x, 1, 1), 0.0)
    s = s + jnp.where(kf >= 2.0, pltpu.roll(s, 2, 1), 0.0)
    s = s + jnp.where(kf >= 4.0, pltpu.roll(s, 4, 1), 0.0)
    return s


def _grp_last_bcast(c, kf):
    n = c.shape[1]
    d = jnp.where(kf == 7.0, c, 0.0)
    d = d + pltpu.roll(d, n - 1, 1)
    d = d + pltpu.roll(d, n - 2, 1)
    d = d + pltpu.roll(d, n - 4, 1)
    return d


def _spline_body(x_ref, w0_ref, b0_ref, w1_ref, b1_ref, w2_ref, b2_ref,
                 e12t_ref, ee_ref, bb_ref, sel_ref, out_ref, ld_ref):
    f32 = jnp.float32
    x = x_ref[...]
    xe = _dot01(x, e12t_ref[...])
    x1 = xe[:, :_NCH]
    x2 = xe[:, _NCH:]

    h = jnp.dot(x2, w0_ref[...], preferred_element_type=f32,
                precision=_PD) + b0_ref[...]
    h = jnp.maximum(h, 0.0)
    h = jnp.dot(h, w1_ref[...], preferred_element_type=f32,
                precision=_PD) + b1_ref[...]
    h = jnp.maximum(h, 0.0)
    rwhd = jnp.dot(h, w2_ref[...], preferred_element_type=f32,
                   precision=_PD) + b2_ref[...]
    rwh = rwhd[:, :2 * _GL]
    rd = rwhd[:, 2 * _GL:]

    kf2 = lax.broadcasted_iota(jnp.int32, rwh.shape, 1)
    kf2 = (kf2 % _NUM_BINS).astype(f32)
    kf = kf2[:, :_GL]

    ewh = jnp.exp(rwh - jnp.max(rwh, axis=1, keepdims=True))
    cewh = _grp_cumsum(ewh, kf2)
    inv_den = 1.0 / _grp_last_bcast(cewh, kf2)
    scale = _SPAN * (1.0 - _MINW * _NUM_BINS)
    wh = _SPAN * _MINW + scale * ewh * inv_den
    cumwh = _SPAN * _MINW * (kf2 + 1.0) + scale * cewh * inv_den
    widths = wh[:, :_GL]
    heights = wh[:, _GL:]
    cumw = cumwh[:, :_GL]
    cumh = cumwh[:, _GL:]
    derivs = _softplus(rd + _DCONST) + _MIND
    derivs = jnp.where(kf == float(_NUM_BINS - 1), 1.0, derivs)

    out_mask = (x1 <= _BLIM) | (x1 >= _ULIM)
    x_in = jnp.where(out_mask, 0.0, x1)
    bb = bb_ref[...]
    xb = _dot01(x_in, bb)
    mask_b = jnp.dot(jnp.where(out_mask, 1.0, 0.0).astype(jnp.bfloat16),
                     bb, preferred_element_type=f32, precision=_PD)

    edges = _LEFT + cumw
    ind = jnp.where(edges <= xb, 1.0, 0.0)
    indp = pltpu.roll(ind, 1, 1)
    indp = jnp.where(kf == 0.0, 1.0, indp)
    oh = indp * jnp.where(kf == float(_NUM_BINS - 1), 1.0, 1.0 - ind)

    xk = edges - widths
    yk = _LEFT + cumh - heights
    dk = jnp.where(kf == 0.0, 1.0, pltpu.roll(derivs, 1, 1))
    dk1 = derivs
    sk = heights / widths
    eps = jnp.clip((xb - xk) / widths, 0.0, 1.0)
    eps_term = eps * (1.0 - eps)
    eps2 = eps * eps
    beta = sk + (dk1 + dk - 2.0 * sk) * eps_term
    alpha = heights * (sk * eps2 + dk * eps_term)
    y_cand = yk + alpha / beta
    dxb_arg = dk1 * eps2 + 2.0 * sk * eps_term + dk * (1.0 - eps) * (1.0 - eps)
    ld_cand = jnp.log(sk * sk * dxb_arg / (beta * beta))

    cat = jnp.concatenate([y_cand * oh, ld_cand * (oh * (1.0 - mask_b))],
                          axis=1)
    res = _dot01(cat, sel_ref[...])
    y1 = jnp.where(out_mask, x1, res[:, :_NCH])

    out_ref[...] = _dot01(jnp.concatenate([y1, x2], axis=1), ee_ref[...])
    ld_ref[...] = res[:, _NCH:_NCH + 1]


def _constants():
    e12t = np.zeros((32, 32), np.float32)
    ee = np.zeros((32, 32), np.float32)
    for j in range(_NCH):
        e12t[2 * j, j] = 1.0
        e12t[2 * j + 1, _NCH + j] = 1.0
        ee[j, 2 * j] = 1.0
        ee[_NCH + j, 2 * j + 1] = 1.0
    bb = np.zeros((_NCH, _GL), np.float32)
    for j in range(_NCH):
        bb[j, j * _NUM_BINS:(j + 1) * _NUM_BINS] = 1.0
    p = bb.T.copy()
    sel = np.zeros((2 * _GL, 32), np.float32)
    sel[:_GL, :_NCH] = p
    sel[_GL:, _NCH] = 1.0
    return e12t, ee, bb, sel


def kernel(x, W0, b0, W1, b1, W2, b2):
    batch = x.shape[0]
    bt = min(1024, batch)
    grid = batch // bt
    mlp_dim = W0.shape[1]

    w2r = W2.reshape(mlp_dim, _NCH, 3 * _NUM_BINS - 1)
    w2w = w2r[:, :, :_NUM_BINS].reshape(mlp_dim, _GL)
    w2h = w2r[:, :, _NUM_BINS:2 * _NUM_BINS].reshape(mlp_dim, _GL)
    w2d = jnp.pad(w2r[:, :, 2 * _NUM_BINS:],
                  ((0, 0), (0, 0), (0, 1))).reshape(mlp_dim, _GL)
    w2all = jnp.concatenate([w2w, w2h, w2d], axis=1)
    b2r = b2.reshape(_NCH, 3 * _NUM_BINS - 1)
    b2w = b2r[:, :_NUM_BINS].reshape(1, _GL)
    b2h = b2r[:, _NUM_BINS:2 * _NUM_BINS].reshape(1, _GL)
    b2d = jnp.pad(b2r[:, 2 * _NUM_BINS:], ((0, 0), (0, 1))).reshape(1, _GL)
    b2all = jnp.concatenate([b2w, b2h, b2d], axis=1)

    e12t, ee, bb, sel = _constants()

    def rep(shape):
        return pl.BlockSpec(shape, lambda i: (0,) * len(shape))

    out, ld = pl.pallas_call(
        _spline_body,
        grid=(grid,),
        in_specs=[
            pl.BlockSpec((bt, 32), lambda i: (i, 0)),
            rep(W0.shape), rep((1, mlp_dim)),
            rep(W1.shape), rep((1, mlp_dim)),
            rep((mlp_dim, 3 * _GL)), rep((1, 3 * _GL)),
            rep(e12t.shape), rep(ee.shape), rep(bb.shape), rep(sel.shape),
        ],
        out_specs=[
            pl.BlockSpec((bt, 32), lambda i: (i, 0)),
            pl.BlockSpec((bt, 1), lambda i: (i, 0)),
        ],
        out_shape=[
            jax.ShapeDtypeStruct((batch, 32), jnp.float32),
            jax.ShapeDtypeStruct((batch, 1), jnp.float32),
        ],
        compiler_params=pltpu.CompilerParams(
            dimension_semantics=("arbitrary",)),
    )(x, W0, b0.reshape(1, mlp_dim), W1, b1.reshape(1, mlp_dim),
      w2all, b2all,
      jnp.asarray(e12t, jnp.bfloat16), jnp.asarray(ee, jnp.bfloat16),
      jnp.asarray(bb, jnp.bfloat16), jnp.asarray(sel, jnp.bfloat16))
    return out, ld.reshape(batch)


# speedup vs baseline: 1.2684x; 1.2684x over previous
import math

import jax
import jax.numpy as jnp
import numpy as np
from jax import lax
from jax.experimental import pallas as pl
from jax.experimental.pallas import tpu as pltpu

_NUM_BINS = 8
_NCH = 16
_GL = _NCH * _NUM_BINS
_LEFT = -1.0
_SPAN = 2.0
_MINW = 1e-4
_MIND = 1e-4
_BLIM = _LEFT + 1e-3
_ULIM = -_LEFT - 1e-3
_DCONST = math.log(math.exp(1.0 - _MIND) - 1.0)
_PD = lax.Precision.DEFAULT


def _softplus(z):
    return jnp.maximum(z, 0.0) + jnp.log(1.0 + jnp.exp(-jnp.abs(z)))


def _dot01(a, m):
    hi = a.astype(jnp.bfloat16)
    lo = (a - hi.astype(jnp.float32)).astype(jnp.bfloat16)
    return (jnp.dot(hi, m, preferred_element_type=jnp.float32, precision=_PD)
            + jnp.dot(lo, m, preferred_element_type=jnp.float32, precision=_PD))


def _spline_body(x_ref, w0_ref, b0_ref, w1_ref, b1_ref, w2_ref, b2_ref,
                 e12t_ref, ee_ref, bb_ref, pp2_ref, bb2_ref, uu_ref,
                 sel_ref, out_ref, ld_ref):
    f32 = jnp.float32
    x = x_ref[...]
    xe = _dot01(x, e12t_ref[...])
    x1 = xe[:, :_NCH]
    x2 = xe[:, _NCH:]

    h = jnp.dot(x2, w0_ref[...], preferred_element_type=f32,
                precision=_PD) + b0_ref[...]
    h = jnp.maximum(h, 0.0)
    h = jnp.dot(h, w1_ref[...], preferred_element_type=f32,
                precision=_PD) + b1_ref[...]
    h = jnp.maximum(h, 0.0)
    rwhd = jnp.dot(h, w2_ref[...], preferred_element_type=f32,
                   precision=_PD) + b2_ref[...]
    rwh = rwhd[:, :2 * _GL]
    rd = rwhd[:, 2 * _GL:]

    kf2 = lax.broadcasted_iota(jnp.int32, rwh.shape, 1)
    kf2 = (kf2 % _NUM_BINS).astype(f32)
    kf = kf2[:, :_GL]

    ewh = jnp.exp(rwh - jnp.max(rwh, axis=1, keepdims=True))
    cewh = _dot01(ewh, uu_ref[...])
    inv_den = 1.0 / _dot01(_dot01(ewh, pp2_ref[...]), bb2_ref[...])
    scale = _SPAN * (1.0 - _MINW * _NUM_BINS)
    wh = _SPAN * _MINW + scale * ewh * inv_den
    cumwh = _SPAN * _MINW * (kf2 + 1.0) + scale * cewh * inv_den
    widths = wh[:, :_GL]
    heights = wh[:, _GL:]
    cumw = cumwh[:, :_GL]
    cumh = cumwh[:, _GL:]
    derivs = _softplus(rd + _DCONST) + _MIND
    derivs = jnp.where(kf == float(_NUM_BINS - 1), 1.0, derivs)

    out_mask = (x1 <= _BLIM) | (x1 >= _ULIM)
    x_in = jnp.where(out_mask, 0.0, x1)
    bb = bb_ref[...]
    xb = _dot01(x_in, bb)
    mask_b = jnp.dot(jnp.where(out_mask, 1.0, 0.0).astype(jnp.bfloat16),
                     bb, preferred_element_type=f32, precision=_PD)

    edges = _LEFT + cumw
    ind = jnp.where(edges <= xb, 1.0, 0.0)
    indp = pltpu.roll(ind, 1, 1)
    indp = jnp.where(kf == 0.0, 1.0, indp)
    oh = indp * jnp.where(kf == float(_NUM_BINS - 1), 1.0, 1.0 - ind)

    xk = edges - widths
    yk = _LEFT + cumh - heights
    dk = jnp.where(kf == 0.0, 1.0, pltpu.roll(derivs, 1, 1))
    dk1 = derivs
    sk = heights / widths
    eps = jnp.clip((xb - xk) / widths, 0.0, 1.0)
    eps_term = eps * (1.0 - eps)
    eps2 = eps * eps
    beta = sk + (dk1 + dk - 2.0 * sk) * eps_term
    alpha = heights * (sk * eps2 + dk * eps_term)
    y_cand = yk + alpha / beta
    dxb_arg = dk1 * eps2 + 2.0 * sk * eps_term + dk * (1.0 - eps) * (1.0 - eps)
    ld_cand = jnp.log(sk * sk * dxb_arg / (beta * beta))

    cat = jnp.concatenate([y_cand * oh, ld_cand * (oh * (1.0 - mask_b))],
                          axis=1)
    res = _dot01(cat, sel_ref[...])
    y1 = jnp.where(out_mask, x1, res[:, :_NCH])

    out_ref[...] = _dot01(jnp.concatenate([y1, x2], axis=1), ee_ref[...])
    ld_ref[...] = res[:, _NCH:_NCH + 1]


def _constants():
    e12t = np.zeros((32, 32), np.float32)
    ee = np.zeros((32, 32), np.float32)
    for j in range(_NCH):
        e12t[2 * j, j] = 1.0
        e12t[2 * j + 1, _NCH + j] = 1.0
        ee[j, 2 * j] = 1.0
        ee[_NCH + j, 2 * j + 1] = 1.0
    bb = np.zeros((_NCH, _GL), np.float32)
    for j in range(_NCH):
        bb[j, j * _NUM_BINS:(j + 1) * _NUM_BINS] = 1.0
    p = bb.T.copy()
    pp2 = np.zeros((2 * _GL, 32), np.float32)
    pp2[:_GL, :_NCH] = p
    pp2[_GL:, _NCH:] = p
    bb2 = np.zeros((32, 2 * _GL), np.float32)
    bb2[:_NCH, :_GL] = bb
    bb2[_NCH:, _GL:] = bb
    u = np.zeros((_GL, _GL), np.float32)
    for a in range(_GL):
        for b in range(_GL):
            if a // _NUM_BINS == b // _NUM_BINS and a <= b:
                u[a, b] = 1.0
    uu = np.zeros((2 * _GL, 2 * _GL), np.float32)
    uu[:_GL, :_GL] = u
    uu[_GL:, _GL:] = u
    sel = np.zeros((2 * _GL, 32), np.float32)
    sel[:_GL, :_NCH] = p
    sel[_GL:, _NCH] = 1.0
    return e12t, ee, bb, pp2, bb2, uu, sel


def kernel(x, W0, b0, W1, b1, W2, b2):
    batch = x.shape[0]
    bt = min(1024, batch)
    grid = batch // bt
    mlp_dim = W0.shape[1]

    w2r = W2.reshape(mlp_dim, _NCH, 3 * _NUM_BINS - 1)
    w2w = w2r[:, :, :_NUM_BINS].reshape(mlp_dim, _GL)
    w2h = w2r[:, :, _NUM_BINS:2 * _NUM_BINS].reshape(mlp_dim, _GL)
    w2d = jnp.pad(w2r[:, :, 2 * _NUM_BINS:],
                  ((0, 0), (0, 0), (0, 1))).reshape(mlp_dim, _GL)
    w2all = jnp.concatenate([w2w, w2h, w2d], axis=1)
    b2r = b2.reshape(_NCH, 3 * _NUM_BINS - 1)
    b2w = b2r[:, :_NUM_BINS].reshape(1, _GL)
    b2h = b2r[:, _NUM_BINS:2 * _NUM_BINS].reshape(1, _GL)
    b2d = jnp.pad(b2r[:, 2 * _NUM_BINS:], ((0, 0), (0, 1))).reshape(1, _GL)
    b2all = jnp.concatenate([b2w, b2h, b2d], axis=1)

    e12t, ee, bb, pp2, bb2, uu, sel = _constants()

    def rep(shape):
        return pl.BlockSpec(shape, lambda i: (0,) * len(shape))

    out, ld = pl.pallas_call(
        _spline_body,
        grid=(grid,),
        in_specs=[
            pl.BlockSpec((bt, 32), lambda i: (i, 0)),
            rep(W0.shape), rep((1, mlp_dim)),
            rep(W1.shape), rep((1, mlp_dim)),
            rep((mlp_dim, 3 * _GL)), rep((1, 3 * _GL)),
            rep(e12t.shape), rep(ee.shape), rep(bb.shape),
            rep(pp2.shape), rep(bb2.shape), rep(uu.shape), rep(sel.shape),
        ],
        out_specs=[
            pl.BlockSpec((bt, 32), lambda i: (i, 0)),
            pl.BlockSpec((bt, 1), lambda i: (i, 0)),
        ],
        out_shape=[
            jax.ShapeDtypeStruct((batch, 32), jnp.float32),
            jax.ShapeDtypeStruct((batch, 1), jnp.float32),
        ],
        compiler_params=pltpu.CompilerParams(
            dimension_semantics=("arbitrary",)),
    )(x, W0, b0.reshape(1, mlp_dim), W1, b1.reshape(1, mlp_dim),
      w2all, b2all,
      jnp.asarray(e12t, jnp.bfloat16), jnp.asarray(ee, jnp.bfloat16),
      jnp.asarray(bb, jnp.bfloat16), jnp.asarray(pp2, jnp.bfloat16),
      jnp.asarray(bb2, jnp.bfloat16), jnp.asarray(uu, jnp.bfloat16),
      jnp.asarray(sel, jnp.bfloat16))
    return out, ld.reshape(batch)


# speedup vs baseline: 1.3938x; 1.0989x over previous
import math

import jax
import jax.numpy as jnp
import numpy as np
from jax import lax
from jax.experimental import pallas as pl
from jax.experimental.pallas import tpu as pltpu

_NUM_BINS = 8
_NCH = 16
_GL = _NCH * _NUM_BINS
_LEFT = -1.0
_SPAN = 2.0
_MINW = 1e-4
_MIND = 1e-4
_BLIM = _LEFT + 1e-3
_ULIM = -_LEFT - 1e-3
_DCONST = math.log(math.exp(1.0 - _MIND) - 1.0)
_PD = lax.Precision.DEFAULT


def _softplus(z):
    return jnp.maximum(z, 0.0) + jnp.log(1.0 + jnp.exp(-jnp.abs(z)))


def _dot01(a, m):
    hi = a.astype(jnp.bfloat16)
    lo = (a - hi.astype(jnp.float32)).astype(jnp.bfloat16)
    return (jnp.dot(hi, m, preferred_element_type=jnp.float32, precision=_PD)
            + jnp.dot(lo, m, preferred_element_type=jnp.float32, precision=_PD))


def _spline_body(x_ref, w0_ref, b0_ref, w1_ref, b1_ref, w2_ref, b2_ref,
                 bbx_ref, pp2_ref, bb2_ref, uu_ref,
                 sel_ref, out_ref, ld_ref):
    f32 = jnp.float32
    x = x_ref[...]

    h = jnp.dot(x, w0_ref[...], preferred_element_type=f32,
                precision=_PD) + b0_ref[...]
    h = jnp.maximum(h, 0.0)
    h = jnp.dot(h, w1_ref[...], preferred_element_type=f32,
                precision=_PD) + b1_ref[...]
    h = jnp.maximum(h, 0.0)
    rwhd = jnp.dot(h, w2_ref[...], preferred_element_type=f32,
                   precision=_PD) + b2_ref[...]
    rwh = rwhd[:, :2 * _GL]
    rd = rwhd[:, 2 * _GL:]

    kf2 = lax.broadcasted_iota(jnp.int32, rwh.shape, 1)
    kf2 = (kf2 % _NUM_BINS).astype(f32)
    kf = kf2[:, :_GL]

    ewh = jnp.exp(rwh - jnp.max(rwh, axis=1, keepdims=True))
    cewh = _dot01(ewh, uu_ref[...])
    inv_den = 1.0 / _dot01(_dot01(ewh, pp2_ref[...]), bb2_ref[...])
    scale = _SPAN * (1.0 - _MINW * _NUM_BINS)
    wh = _SPAN * _MINW + scale * ewh * inv_den
    cumwh = _SPAN * _MINW * (kf2 + 1.0) + scale * cewh * inv_den
    widths = wh[:, :_GL]
    heights = wh[:, _GL:]
    cumw = cumwh[:, :_GL]
    cumh = cumwh[:, _GL:]
    derivs = _softplus(rd + _DCONST) + _MIND
    derivs = jnp.where(kf == float(_NUM_BINS - 1), 1.0, derivs)

    xrawb = _dot01(x, bbx_ref[...])
    mask_b = (xrawb <= _BLIM) | (xrawb >= _ULIM)
    xb = jnp.where(mask_b, 0.0, xrawb)

    edges = _LEFT + cumw
    ind = jnp.where(edges <= xb, 1.0, 0.0)
    indp = pltpu.roll(ind, 1, 1)
    indp = jnp.where(kf == 0.0, 1.0, indp)
    oh = indp * jnp.where(kf == float(_NUM_BINS - 1), 1.0, 1.0 - ind)

    xk = edges - widths
    yk = _LEFT + cumh - heights
    dk = jnp.where(kf == 0.0, 1.0, pltpu.roll(derivs, 1, 1))
    dk1 = derivs
    sk = heights / widths
    eps = jnp.clip((xb - xk) / widths, 0.0, 1.0)
    eps_term = eps * (1.0 - eps)
    eps2 = eps * eps
    beta = sk + (dk1 + dk - 2.0 * sk) * eps_term
    alpha = heights * (sk * eps2 + dk * eps_term)
    y_cand = yk + alpha / beta
    dxb_arg = dk1 * eps2 + 2.0 * sk * eps_term + dk * (1.0 - eps) * (1.0 - eps)
    ld_cand = jnp.log(sk * sk * dxb_arg / (beta * beta))

    y_masked = jnp.where(mask_b, xrawb, y_cand)
    ld_masked = jnp.where(mask_b, 0.0, ld_cand)
    cat = jnp.concatenate([y_masked * oh, ld_masked * oh], axis=1)
    res = _dot01(cat, sel_ref[...])
    col_odd = lax.broadcasted_iota(jnp.int32, x.shape, 1) % 2
    out_ref[...] = res[:, :32] + jnp.where(col_odd == 1, x, 0.0)
    ld_ref[...] = res[:, 32:33]


def _constants():
    bbx = np.zeros((32, _GL), np.float32)
    for j in range(_NCH):
        bbx[2 * j, j * _NUM_BINS:(j + 1) * _NUM_BINS] = 1.0
    p = np.zeros((_GL, _NCH), np.float32)
    for j in range(_NCH):
        p[j * _NUM_BINS:(j + 1) * _NUM_BINS, j] = 1.0
    pp2 = np.zeros((2 * _GL, 32), np.float32)
    pp2[:_GL, :_NCH] = p
    pp2[_GL:, _NCH:] = p
    bb2 = np.zeros((32, 2 * _GL), np.float32)
    bb2[:_NCH, :_GL] = bbx[0::2, :]
    bb2[_NCH:, _GL:] = bbx[0::2, :]
    u = np.zeros((_GL, _GL), np.float32)
    for a in range(_GL):
        for b in range(_GL):
            if a // _NUM_BINS == b // _NUM_BINS and a <= b:
                u[a, b] = 1.0
    uu = np.zeros((2 * _GL, 2 * _GL), np.float32)
    uu[:_GL, :_GL] = u
    uu[_GL:, _GL:] = u
    sel = np.zeros((2 * _GL, 33), np.float32)
    for j in range(_NCH):
        sel[j * _NUM_BINS:(j + 1) * _NUM_BINS, 2 * j] = 1.0
    sel[_GL:, 32] = 1.0
    return bbx, pp2, bb2, uu, sel


def kernel(x, W0, b0, W1, b1, W2, b2):
    batch = x.shape[0]
    bt = min(1024, batch)
    grid = batch // bt
    mlp_dim = W0.shape[1]

    w2r = W2.reshape(mlp_dim, _NCH, 3 * _NUM_BINS - 1)
    w2w = w2r[:, :, :_NUM_BINS].reshape(mlp_dim, _GL)
    w2h = w2r[:, :, _NUM_BINS:2 * _NUM_BINS].reshape(mlp_dim, _GL)
    w2d = jnp.pad(w2r[:, :, 2 * _NUM_BINS:],
                  ((0, 0), (0, 0), (0, 1))).reshape(mlp_dim, _GL)
    w2all = jnp.concatenate([w2w, w2h, w2d], axis=1)
    b2r = b2.reshape(_NCH, 3 * _NUM_BINS - 1)
    b2w = b2r[:, :_NUM_BINS].reshape(1, _GL)
    b2h = b2r[:, _NUM_BINS:2 * _NUM_BINS].reshape(1, _GL)
    b2d = jnp.pad(b2r[:, 2 * _NUM_BINS:], ((0, 0), (0, 1))).reshape(1, _GL)
    b2all = jnp.concatenate([b2w, b2h, b2d], axis=1)

    bbx, pp2, bb2, uu, sel = _constants()
    w0x = jnp.zeros((32, mlp_dim), jnp.float32).at[1::2, :].set(W0)

    def rep(shape):
        return pl.BlockSpec(shape, lambda i: (0,) * len(shape))

    out, ld = pl.pallas_call(
        _spline_body,
        grid=(grid,),
        in_specs=[
            pl.BlockSpec((bt, 32), lambda i: (i, 0)),
            rep((32, mlp_dim)), rep((1, mlp_dim)),
            rep(W1.shape), rep((1, mlp_dim)),
            rep((mlp_dim, 3 * _GL)), rep((1, 3 * _GL)),
            rep(bbx.shape), rep(pp2.shape), rep(bb2.shape),
            rep(uu.shape), rep(sel.shape),
        ],
        out_specs=[
            pl.BlockSpec((bt, 32), lambda i: (i, 0)),
            pl.BlockSpec((bt, 1), lambda i: (i, 0)),
        ],
        out_shape=[
            jax.ShapeDtypeStruct((batch, 32), jnp.float32),
            jax.ShapeDtypeStruct((batch, 1), jnp.float32),
        ],
        compiler_params=pltpu.CompilerParams(
            dimension_semantics=("arbitrary",)),
    )(x, w0x, b0.reshape(1, mlp_dim), W1, b1.reshape(1, mlp_dim),
      w2all, b2all,
      jnp.asarray(bbx, jnp.bfloat16), jnp.asarray(pp2, jnp.bfloat16),
      jnp.asarray(bb2, jnp.bfloat16), jnp.asarray(uu, jnp.bfloat16),
      jnp.asarray(sel, jnp.bfloat16))
    return out, ld.reshape(batch)


# speedup vs baseline: 1.5205x; 1.0909x over previous
import math

import jax
import jax.numpy as jnp
import numpy as np
from jax import lax
from jax.experimental import pallas as pl
from jax.experimental.pallas import tpu as pltpu

_NUM_BINS = 8
_NCH = 16
_GL = _NCH * _NUM_BINS
_LEFT = -1.0
_SPAN = 2.0
_MINW = 1e-4
_MIND = 1e-4
_BLIM = _LEFT + 1e-3
_ULIM = -_LEFT - 1e-3
_DCONST = math.log(math.exp(1.0 - _MIND) - 1.0)
_PD = lax.Precision.DEFAULT


def _softplus(z):
    return jnp.maximum(z, 0.0) + jnp.log(1.0 + jnp.exp(-jnp.abs(z)))


def _dot01(a, m):
    hi = a.astype(jnp.bfloat16)
    lo = (a - hi.astype(jnp.float32)).astype(jnp.bfloat16)
    return (jnp.dot(hi, m, preferred_element_type=jnp.float32, precision=_PD)
            + jnp.dot(lo, m, preferred_element_type=jnp.float32, precision=_PD))


def _spline_body(x_ref, w0_ref, b0_ref, w1_ref, b1_ref, w2_ref, b2_ref,
                 bbx_ref, uu_ref, sel_ref, out_ref, ld_ref):
    f32 = jnp.float32
    x = x_ref[...]

    h = jnp.dot(x.astype(jnp.bfloat16), w0_ref[...],
                preferred_element_type=f32, precision=_PD) + b0_ref[...]
    h = jnp.maximum(h, 0.0).astype(jnp.bfloat16)
    h = jnp.dot(h, w1_ref[...], preferred_element_type=f32,
                precision=_PD) + b1_ref[...]
    h = jnp.maximum(h, 0.0).astype(jnp.bfloat16)
    rwhd = jnp.dot(h, w2_ref[...], preferred_element_type=f32,
                   precision=_PD) + b2_ref[...]
    rwh = rwhd[:, :2 * _GL]
    rd = rwhd[:, 2 * _GL:]

    kf2 = lax.broadcasted_iota(jnp.int32, rwh.shape, 1)
    kf2 = (kf2 % _NUM_BINS).astype(f32)
    kf = kf2[:, :_GL]

    ewh = jnp.exp(rwh - jnp.max(rwh, axis=1, keepdims=True))
    cg = _dot01(ewh, uu_ref[...])
    cewh = cg[:, :2 * _GL]
    inv_den = 1.0 / cg[:, 2 * _GL:]
    scale = _SPAN * (1.0 - _MINW * _NUM_BINS)
    wh = _SPAN * _MINW + scale * ewh * inv_den
    cumwh = _SPAN * _MINW * (kf2 + 1.0) + scale * cewh * inv_den
    widths = wh[:, :_GL]
    heights = wh[:, _GL:]
    cumw = cumwh[:, :_GL]
    cumh = cumwh[:, _GL:]
    derivs = _softplus(rd + _DCONST) + _MIND
    derivs = jnp.where(kf == float(_NUM_BINS - 1), 1.0, derivs)

    xrawb = _dot01(x, bbx_ref[...])
    mask_b = (xrawb <= _BLIM) | (xrawb >= _ULIM)
    xb = jnp.where(mask_b, 0.0, xrawb)

    edges = _LEFT + cumw
    ind = jnp.where(edges <= xb, 1.0, 0.0)
    indp = pltpu.roll(ind, 1, 1)
    indp = jnp.where(kf == 0.0, 1.0, indp)
    oh = indp * jnp.where(kf == float(_NUM_BINS - 1), 1.0, 1.0 - ind)

    xk = edges - widths
    yk = _LEFT + cumh - heights
    dk = jnp.where(kf == 0.0, 1.0, pltpu.roll(derivs, 1, 1))
    dk1 = derivs
    sk = heights / widths
    eps = jnp.clip((xb - xk) / widths, 0.0, 1.0)
    eps_term = eps * (1.0 - eps)
    eps2 = eps * eps
    beta = sk + (dk1 + dk - 2.0 * sk) * eps_term
    alpha = heights * (sk * eps2 + dk * eps_term)
    y_cand = yk + alpha / beta
    dxb_arg = dk1 * eps2 + 2.0 * sk * eps_term + dk * (1.0 - eps) * (1.0 - eps)
    ld_cand = jnp.log(sk * sk * dxb_arg / (beta * beta))

    y_masked = jnp.where(mask_b, xrawb, y_cand)
    ld_masked = jnp.where(mask_b, 0.0, ld_cand)
    cat = jnp.concatenate([y_masked * oh, ld_masked * oh], axis=1)
    res = _dot01(cat, sel_ref[...])
    col_odd = lax.broadcasted_iota(jnp.int32, x.shape, 1) % 2
    out_ref[...] = res[:, :32] + jnp.where(col_odd == 1, x, 0.0)
    ld_ref[...] = res[:, 32:33]


def _constants():
    bbx = np.zeros((32, _GL), np.float32)
    for j in range(_NCH):
        bbx[2 * j, j * _NUM_BINS:(j + 1) * _NUM_BINS] = 1.0
    p = np.zeros((_GL, _NCH), np.float32)
    for j in range(_NCH):
        p[j * _NUM_BINS:(j + 1) * _NUM_BINS, j] = 1.0
    u = np.zeros((_GL, _GL), np.float32)
    g = np.zeros((_GL, _GL), np.float32)
    for a in range(_GL):
        for b in range(_GL):
            if a // _NUM_BINS == b // _NUM_BINS:
                g[a, b] = 1.0
                if a <= b:
                    u[a, b] = 1.0
    uu = np.zeros((2 * _GL, 4 * _GL), np.float32)
    uu[:_GL, :_GL] = u
    uu[_GL:, _GL:2 * _GL] = u
    uu[:_GL, 2 * _GL:3 * _GL] = g
    uu[_GL:, 3 * _GL:] = g
    sel = np.zeros((2 * _GL, 33), np.float32)
    for j in range(_NCH):
        sel[j * _NUM_BINS:(j + 1) * _NUM_BINS, 2 * j] = 1.0
    sel[_GL:, 32] = 1.0
    return bbx, uu, sel


def kernel(x, W0, b0, W1, b1, W2, b2):
    batch = x.shape[0]
    bt = min(2048, batch)
    grid = batch // bt
    mlp_dim = W0.shape[1]

    w2r = W2.reshape(mlp_dim, _NCH, 3 * _NUM_BINS - 1)
    w2w = w2r[:, :, :_NUM_BINS].reshape(mlp_dim, _GL)
    w2h = w2r[:, :, _NUM_BINS:2 * _NUM_BINS].reshape(mlp_dim, _GL)
    w2d = jnp.pad(w2r[:, :, 2 * _NUM_BINS:],
                  ((0, 0), (0, 0), (0, 1))).reshape(mlp_dim, _GL)
    w2all = jnp.concatenate([w2w, w2h, w2d], axis=1)
    b2r = b2.reshape(_NCH, 3 * _NUM_BINS - 1)
    b2w = b2r[:, :_NUM_BINS].reshape(1, _GL)
    b2h = b2r[:, _NUM_BINS:2 * _NUM_BINS].reshape(1, _GL)
    b2d = jnp.pad(b2r[:, 2 * _NUM_BINS:], ((0, 0), (0, 1))).reshape(1, _GL)
    b2all = jnp.concatenate([b2w, b2h, b2d], axis=1)

    bbx, uu, sel = _constants()
    w0x = jnp.zeros((32, mlp_dim), jnp.float32).at[1::2, :].set(W0)

    def rep(shape):
        return pl.BlockSpec(shape, lambda i: (0,) * len(shape))

    out, ld = pl.pallas_call(
        _spline_body,
        grid=(grid,),
        in_specs=[
            pl.BlockSpec((bt, 32), lambda i: (i, 0)),
            rep((32, mlp_dim)), rep((1, mlp_dim)),
            rep(W1.shape), rep((1, mlp_dim)),
            rep((mlp_dim, 3 * _GL)), rep((1, 3 * _GL)),
            rep(bbx.shape), rep(uu.shape), rep(sel.shape),
        ],
        out_specs=[
            pl.BlockSpec((bt, 32), lambda i: (i, 0)),
            pl.BlockSpec((bt, 1), lambda i: (i, 0)),
        ],
        out_shape=[
            jax.ShapeDtypeStruct((batch, 32), jnp.float32),
            jax.ShapeDtypeStruct((batch, 1), jnp.float32),
        ],
        compiler_params=pltpu.CompilerParams(
            dimension_semantics=("arbitrary",)),
    )(x, w0x.astype(jnp.bfloat16), b0.reshape(1, mlp_dim),
      W1.astype(jnp.bfloat16), b1.reshape(1, mlp_dim),
      w2all.astype(jnp.bfloat16), b2all,
      jnp.asarray(bbx, jnp.bfloat16), jnp.asarray(uu, jnp.bfloat16),
      jnp.asarray(sel, jnp.bfloat16))
    return out, ld.reshape(batch)


# speedup vs baseline: 1.5214x; 1.0005x over previous
import math

import jax
import jax.numpy as jnp
import numpy as np
from jax import lax
from jax.experimental import pallas as pl
from jax.experimental.pallas import tpu as pltpu

_NUM_BINS = 8
_NCH = 16
_GL = _NCH * _NUM_BINS
_LEFT = -1.0
_SPAN = 2.0
_MINW = 1e-4
_MIND = 1e-4
_BLIM = _LEFT + 1e-3
_ULIM = -_LEFT - 1e-3
_DCONST = math.log(math.exp(1.0 - _MIND) - 1.0)
_PD = lax.Precision.DEFAULT


def _softplus(z):
    return jnp.maximum(z, 0.0) + jnp.log(1.0 + jnp.exp(-jnp.abs(z)))


def _dot01(a, m):
    hi = a.astype(jnp.bfloat16)
    lo = (a - hi.astype(jnp.float32)).astype(jnp.bfloat16)
    return (jnp.dot(hi, m, preferred_element_type=jnp.float32, precision=_PD)
            + jnp.dot(lo, m, preferred_element_type=jnp.float32, precision=_PD))


def _spline_body(x_ref, w0_ref, b0_ref, w1_ref, b1_ref, w2_ref, b2_ref,
                 bbx_ref, uu_ref, sel_ref, out_ref, ld_ref):
    f32 = jnp.float32
    x = x_ref[...]

    h = jnp.dot(x.astype(jnp.bfloat16), w0_ref[...],
                preferred_element_type=f32, precision=_PD) + b0_ref[...]
    h = jnp.maximum(h, 0.0).astype(jnp.bfloat16)
    h = jnp.dot(h, w1_ref[...], preferred_element_type=f32,
                precision=_PD) + b1_ref[...]
    h = jnp.maximum(h, 0.0).astype(jnp.bfloat16)
    rwhd = jnp.dot(h, w2_ref[...], preferred_element_type=f32,
                   precision=_PD) + b2_ref[...]
    rwh = rwhd[:, :2 * _GL]
    rd = rwhd[:, 2 * _GL:]

    kf2 = lax.broadcasted_iota(jnp.int32, (1, 2 * _GL), 1)
    kf2 = (kf2 % _NUM_BINS).astype(f32)
    kf = kf2[:, :_GL]

    ewh = jnp.exp(rwh - jnp.max(rwh, axis=1, keepdims=True))
    cg = _dot01(ewh, uu_ref[...])
    cewh = cg[:, :2 * _GL]
    inv_den = 1.0 / cg[:, 2 * _GL:]
    scale = _SPAN * (1.0 - _MINW * _NUM_BINS)
    wh = _SPAN * _MINW + scale * ewh * inv_den
    cumwh = _SPAN * _MINW * (kf2 + 1.0) + scale * cewh * inv_den
    widths = wh[:, :_GL]
    heights = wh[:, _GL:]
    cumw = cumwh[:, :_GL]
    cumh = cumwh[:, _GL:]
    derivs = _softplus(rd + _DCONST) + _MIND
    derivs = jnp.where(kf == float(_NUM_BINS - 1), 1.0, derivs)

    xrawb = _dot01(x, bbx_ref[...])
    mask_b = (xrawb <= _BLIM) | (xrawb >= _ULIM)
    xb = jnp.where(mask_b, 0.0, xrawb)

    edges = _LEFT + cumw
    ind = jnp.where(edges <= xb, 1.0, 0.0)
    indp = pltpu.roll(ind, 1, 1)
    indp = jnp.where(kf == 0.0, 1.0, indp)
    oh = indp * jnp.where(kf == float(_NUM_BINS - 1), 1.0, 1.0 - ind)

    xk = edges - widths
    yk = _LEFT + cumh - heights
    dk = jnp.where(kf == 0.0, 1.0, pltpu.roll(derivs, 1, 1))
    dk1 = derivs
    sk = heights / widths
    eps = jnp.clip((xb - xk) / widths, 0.0, 1.0)
    om = 1.0 - eps
    eps_term = eps * om
    eps2 = eps * eps
    beta = sk + (dk1 + dk - 2.0 * sk) * eps_term
    alpha = heights * (sk * eps2 + dk * eps_term)
    y_cand = yk + alpha / beta
    dxb_arg = dk1 * eps2 + 2.0 * sk * eps_term + dk * om * om
    ld_cand = jnp.log(sk * sk * dxb_arg / (beta * beta))

    y_masked = jnp.where(mask_b, xrawb, y_cand)
    ld_masked = jnp.where(mask_b, 0.0, ld_cand)
    cat = jnp.concatenate([y_masked * oh, ld_masked * oh], axis=1)
    res = _dot01(cat, sel_ref[...])
    col_odd = lax.broadcasted_iota(jnp.int32, (1, 32), 1) % 2
    out_ref[...] = res[:, :32] + jnp.where(col_odd == 1, x, 0.0)
    ld_ref[...] = res[:, 32:33]


def _constants():
    bbx = np.zeros((32, _GL), np.float32)
    for j in range(_NCH):
        bbx[2 * j, j * _NUM_BINS:(j + 1) * _NUM_BINS] = 1.0
    p = np.zeros((_GL, _NCH), np.float32)
    for j in range(_NCH):
        p[j * _NUM_BINS:(j + 1) * _NUM_BINS, j] = 1.0
    u = np.zeros((_GL, _GL), np.float32)
    g = np.zeros((_GL, _GL), np.float32)
    for a in range(_GL):
        for b in range(_GL):
            if a // _NUM_BINS == b // _NUM_BINS:
                g[a, b] = 1.0
                if a <= b:
                    u[a, b] = 1.0
    uu = np.zeros((2 * _GL, 4 * _GL), np.float32)
    uu[:_GL, :_GL] = u
    uu[_GL:, _GL:2 * _GL] = u
    uu[:_GL, 2 * _GL:3 * _GL] = g
    uu[_GL:, 3 * _GL:] = g
    sel = np.zeros((2 * _GL, 33), np.float32)
    for j in range(_NCH):
        sel[j * _NUM_BINS:(j + 1) * _NUM_BINS, 2 * j] = 1.0
    sel[_GL:, 32] = 1.0
    return bbx, uu, sel


def kernel(x, W0, b0, W1, b1, W2, b2):
    batch = x.shape[0]
    bt = min(2048, batch)
    grid = batch // bt
    mlp_dim = W0.shape[1]

    w2r = W2.reshape(mlp_dim, _NCH, 3 * _NUM_BINS - 1)
    w2w = w2r[:, :, :_NUM_BINS].reshape(mlp_dim, _GL)
    w2h = w2r[:, :, _NUM_BINS:2 * _NUM_BINS].reshape(mlp_dim, _GL)
    w2d = jnp.pad(w2r[:, :, 2 * _NUM_BINS:],
                  ((0, 0), (0, 0), (0, 1))).reshape(mlp_dim, _GL)
    w2all = jnp.concatenate([w2w, w2h, w2d], axis=1)
    b2r = b2.reshape(_NCH, 3 * _NUM_BINS - 1)
    b2w = b2r[:, :_NUM_BINS].reshape(1, _GL)
    b2h = b2r[:, _NUM_BINS:2 * _NUM_BINS].reshape(1, _GL)
    b2d = jnp.pad(b2r[:, 2 * _NUM_BINS:], ((0, 0), (0, 1))).reshape(1, _GL)
    b2all = jnp.concatenate([b2w, b2h, b2d], axis=1)

    bbx, uu, sel = _constants()
    w0x = jnp.zeros((32, mlp_dim), jnp.float32).at[1::2, :].set(W0)

    def rep(shape):
        return pl.BlockSpec(shape, lambda i: (0,) * len(shape))

    out, ld = pl.pallas_call(
        _spline_body,
        grid=(grid,),
        in_specs=[
            pl.BlockSpec((bt, 32), lambda i: (i, 0)),
            rep((32, mlp_dim)), rep((1, mlp_dim)),
            rep(W1.shape), rep((1, mlp_dim)),
            rep((mlp_dim, 3 * _GL)), rep((1, 3 * _GL)),
            rep(bbx.shape), rep(uu.shape), rep(sel.shape),
        ],
        out_specs=[
            pl.BlockSpec((bt, 32), lambda i: (i, 0)),
            pl.BlockSpec((bt, 1), lambda i: (i, 0)),
        ],
        out_shape=[
            jax.ShapeDtypeStruct((batch, 32), jnp.float32),
            jax.ShapeDtypeStruct((batch, 1), jnp.float32),
        ],
        compiler_params=pltpu.CompilerParams(
            dimension_semantics=("parallel",)),
    )(x, w0x.astype(jnp.bfloat16), b0.reshape(1, mlp_dim),
      W1.astype(jnp.bfloat16), b1.reshape(1, mlp_dim),
      w2all.astype(jnp.bfloat16), b2all,
      jnp.asarray(bbx, jnp.bfloat16), jnp.asarray(uu, jnp.bfloat16),
      jnp.asarray(sel, jnp.bfloat16))
    return out, ld.reshape(batch)
